# baseline (speedup 1.0000x reference)
import jax
import jax.numpy as jnp
from jax.experimental import pallas as pl

_DILATION = 2


def _slice_kernel(x_ref, o_ref):
    for k in range(o_ref.shape[-1]):
        o_ref[:, :, k] = x_ref[:, :, _DILATION * k]


def kernel(edge_index):
    two, n, kd = edge_index.shape
    k = kd // _DILATION
    blk = 4000
    grid = (two, n // blk)
    return pl.pallas_call(
        _slice_kernel,
        grid=grid,
        in_specs=[pl.BlockSpec((1, blk, kd), lambda i, j: (i, j, 0))],
        out_specs=pl.BlockSpec((1, blk, k), lambda i, j: (i, j, 0)),
        out_shape=jax.ShapeDtypeStruct((two, n, k), edge_index.dtype),
    )(edge_index)


# speedup vs baseline: 1.4366x; 1.4366x over previous
import jax
import jax.numpy as jnp
import numpy as np
from jax.experimental import pallas as pl

_DILATION = 2


def _sel_kernel(x_ref, s_ref, o_ref):
    xf = x_ref[...].astype(jnp.float32)
    y = jax.lax.dot_general(
        xf, s_ref[...], (((1,), (0,)), ((), ())),
        preferred_element_type=jnp.float32,
        precision=jax.lax.Precision.HIGHEST,
    )
    o_ref[...] = y.astype(jnp.int32)


def kernel(edge_index):
    two, n, kd = edge_index.shape
    k = kd // _DILATION
    rows = two * n
    x = edge_index.reshape(rows, kd)
    sel = jnp.asarray(np.eye(kd, dtype=np.float32)[:, ::_DILATION])
    blk = 8000
    grid = (rows // blk,)
    out = pl.pallas_call(
        _sel_kernel,
        grid=grid,
        in_specs=[
            pl.BlockSpec((blk, kd), lambda i: (i, 0)),
            pl.BlockSpec((kd, k), lambda i: (0, 0)),
        ],
        out_specs=pl.BlockSpec((blk, k), lambda i: (i, 0)),
        out_shape=jax.ShapeDtypeStruct((rows, k), edge_index.dtype),
    )(x, sel)
    return out.reshape(two, n, k)
